# baseline (speedup 1.0000x reference)
import jax
import jax.numpy as jnp
from jax import lax
from jax.experimental import pallas as pl
from jax.experimental.pallas import tpu as pltpu
from jax.experimental.pallas import tpu_sc as plsc

_H = 64
_CH = 4096
_SUB = 256
_NTS = 16


def _ceil_to(x, m):
    return -(-x // m) * m


def _seg_cfg(n_dst):
    half = -(-n_dst // 2)
    n_pass = -(-half // 16384)
    nq = _ceil_to(-(-n_dst // (2 * n_pass)), 512)
    return n_pass, nq, 2 * n_pass * nq


def _np_rows(n_dst):
    return _seg_cfg(n_dst)[2]


def _seg_builder(specs, with_cnt):
    cfgs = [_seg_cfg(n_dst) for (_, n_dst, _) in specs]
    max_accr = max(_ceil_to(nq + 16, _SUB) for (_, nq, _) in cfgs)

    mesh = plsc.VectorSubcoreMesh(core_axis_name="c", subcore_axis_name="s")

    out_type = [jax.ShapeDtypeStruct((np_out, _H), jnp.float32)
                for (_, _, np_out) in cfgs]
    if with_cnt:
        out_type += [jax.ShapeDtypeStruct((np_out,), jnp.float32)
                     for (_, _, np_out) in cfgs]

    nsr = _CH // _SUB + 1
    scratch = [
        pltpu.VMEM((_CH,), jnp.int32),
        pltpu.VMEM((_CH,), jnp.int32),
        pltpu.VMEM((nsr, _SUB), jnp.int32),
        pltpu.VMEM((nsr, _SUB), jnp.int32),
        pltpu.VMEM((_SUB, _H), jnp.float32),
        pltpu.VMEM((_SUB, _H), jnp.float32),
        pltpu.VMEM((_SUB, _H), jnp.float32),
        pltpu.VMEM((_SUB,), jnp.float32),
        pltpu.VMEM((_SUB,), jnp.float32),
        pltpu.VMEM_SHARED((max_accr, _H), jnp.float32),
        pltpu.VMEM_SHARED((max_accr,), jnp.float32),
        pltpu.SemaphoreType.DMA,
        pltpu.SemaphoreType.DMA,
        pltpu.SemaphoreType.DMA,
        pltpu.SemaphoreType.DMA,
    ]

    nty = len(specs)

    def body(*refs):
        ins = refs[:3 * nty]
        outs = refs[3 * nty:3 * nty + nty]
        if with_cnt:
            cnt_outs = refs[4 * nty:5 * nty]
            scr = refs[5 * nty:]
        else:
            cnt_outs = [None] * nty
            scr = refs[4 * nty:]
        (s_src, s_dst, k_src, k_dst, rows0, rows1, zrows, zvec, ones,
         acc, cacc, sg0, sg1, ss0, ss1) = scr
        rows_b = (rows0, rows1)
        sg_b = (sg0, sg1)
        ss_b = (ss0, ss1)
        c = lax.axis_index("c")
        s = lax.axis_index("s")

        zero16 = jnp.zeros((16,), jnp.float32)
        one16 = jnp.ones((16,), jnp.float32)

        def zrow_body(i, _):
            for jj in range(_H // 16):
                zrows[i, pl.ds(jj * 16, 16)] = zero16
            return 0
        lax.fori_loop(0, _SUB, zrow_body, 0)

        def zvec_body(i, _):
            zvec[pl.ds(i * 16, 16)] = zero16
            ones[pl.ds(i * 16, 16)] = one16
            return 0
        lax.fori_loop(0, _SUB // 16, zvec_body, 0)

        di = lax.iota(jnp.int32, 16)
        dump_s = s * 16 + di

        def one_pass(p, src_ref, dst_ref, x_ref, out_ref, cnt_ref,
                     n_dst, nq, n_chunks_per_tile, nzc, noc, dump_d):
            q = 2 * p + c
            lo = q * nq
            hi = jnp.minimum(lo + nq, n_dst)

            for k in range(-(-nzc // _NTS)):
                ci = k * _NTS + s

                @pl.when(ci < nzc)
                def _():
                    pltpu.sync_copy(zrows, acc.at[pl.ds(ci * _SUB, _SUB)])
                    if with_cnt:
                        pltpu.sync_copy(zvec,
                                        cacc.at[pl.ds(ci * _SUB, _SUB)])
            plsc.subcore_barrier()

            def chunk_body(ci, _):
                base = (ci * _NTS + s) * _CH
                pltpu.sync_copy(src_ref.at[pl.ds(base, _CH)], s_src)
                pltpu.sync_copy(dst_ref.at[pl.ds(base, _CH)], s_dst)

                def pf(i, _):
                    r = i // (_SUB // 16)
                    g = i % (_SUB // 16)
                    k_src[r, pl.ds(g * 16, 16)] = dump_s
                    k_dst[r, pl.ds(g * 16, 16)] = dump_d
                    return 0
                lax.fori_loop(0, nsr * (_SUB // 16), pf, 0)

                def fl(i, kc):
                    d = s_dst[pl.ds(i * 16, 16)]
                    sr = s_src[pl.ds(i * 16, 16)]
                    m = (d >= lo) & (d < hi)
                    mi = m.astype(jnp.int32)
                    pos = kc + plsc.cumsum(mi) - mi
                    pr = pos // _SUB
                    pc = pos % _SUB
                    plsc.store_scatter(k_src, [pr, pc], sr, mask=m)
                    plsc.store_scatter(k_dst, [pr, pc], d - lo, mask=m)
                    return kc + jnp.sum(mi)
                kc = lax.fori_loop(0, _CH // 16, fl, jnp.int32(0))
                nsub = (kc + _SUB - 1) // _SUB

                @pl.when(nsub > 0)
                def _():
                    pltpu.async_copy(x_ref.at[k_src.at[0]], rows0, sg0)

                def sub2(t, _):
                    for kb in range(2):
                        j = 2 * t + kb
                        rw, sg = rows_b[kb], sg_b[kb]
                        rw_o, sg_o = rows_b[1 - kb], sg_b[1 - kb]

                        @pl.when(j < nsub)
                        def _():
                            @pl.when(j + 1 < nsub)
                            def _():
                                pltpu.async_copy(
                                    x_ref.at[k_src.at[j + 1]], rw_o, sg_o)
                            pltpu.make_async_copy(
                                x_ref.at[k_src.at[j]], rw, sg).wait()
                            pltpu.sync_copy(rw, acc.at[k_dst.at[j]],
                                            add=True)
                            if with_cnt:
                                pltpu.sync_copy(ones, cacc.at[k_dst.at[j]],
                                                add=True)
                    return 0
                lax.fori_loop(0, (nsub + 1) // 2, sub2, 0)
                return 0
            lax.fori_loop(0, n_chunks_per_tile, chunk_body, 0)

            plsc.subcore_barrier()

            for k in range(-(-noc // _NTS)):
                ci = k * _NTS + s

                @pl.when(ci < noc)
                def _():
                    off = ci * _SUB
                    pltpu.sync_copy(acc.at[pl.ds(off, _SUB)],
                                    out_ref.at[pl.ds(lo + off, _SUB)])
                    if with_cnt:
                        pltpu.sync_copy(cacc.at[pl.ds(off, _SUB)],
                                        cnt_ref.at[pl.ds(lo + off, _SUB)])
            plsc.subcore_barrier()

        for t in range(nty):
            _, n_dst, n_ep = specs[t]
            n_pass, nq, _ = cfgs[t]
            accr = _ceil_to(nq + 16, _SUB)
            for p in range(n_pass):
                one_pass(p, ins[3 * t], ins[3 * t + 1], ins[3 * t + 2],
                         outs[t], cnt_outs[t], n_dst, nq,
                         n_ep // (_CH * _NTS), accr // _SUB, nq // _SUB,
                         nq + di)

    return pl.kernel(body, out_type=out_type, mesh=mesh,
                     scratch_types=scratch,
                     compiler_params=pltpu.CompilerParams(
                         needs_layout_passes=False,
                         use_tc_tiling_on_sc=False))


def _upd_builder(np_rows, n_agg):
    blk = 1024
    grid = (np_rows // blk,)

    def body(*refs):
        if n_agg == 2:
            s1, c1, s2, c2, x, w1, w2, wr1, wr2, b1, b2, o = refs
        else:
            s1, c1, x, w1, wr1, b1, o = refs
        xv = x[...]
        t = jnp.dot(s1[...] / jnp.maximum(c1[...], 1.0), w1[...],
                    preferred_element_type=jnp.float32)
        t = t + jnp.dot(xv, wr1[...], preferred_element_type=jnp.float32)
        t = t + b1[...]
        if n_agg == 2:
            t2 = jnp.dot(s2[...] / jnp.maximum(c2[...], 1.0), w2[...],
                         preferred_element_type=jnp.float32)
            t2 = t2 + jnp.dot(xv, wr2[...],
                              preferred_element_type=jnp.float32)
            t = t + (t2 + b2[...])
        o[...] = jnp.maximum(t, 0.0)

    bs_feat = pl.BlockSpec((blk, _H), lambda i: (i, 0))
    bs_cnt = pl.BlockSpec((blk, 1), lambda i: (i, 0))
    bs_w = pl.BlockSpec((_H, _H), lambda i: (0, 0))
    bs_b = pl.BlockSpec((1, _H), lambda i: (0, 0))
    if n_agg == 2:
        in_specs = [bs_feat, bs_cnt, bs_feat, bs_cnt, bs_feat,
                    bs_w, bs_w, bs_w, bs_w, bs_b, bs_b]
    else:
        in_specs = [bs_feat, bs_cnt, bs_feat, bs_w, bs_w, bs_b]
    return pl.pallas_call(
        body, grid=grid, in_specs=in_specs,
        out_specs=pl.BlockSpec((blk, _H), lambda i: (i, 0)),
        out_shape=jax.ShapeDtypeStruct((np_rows, _H), jnp.float32))


def _head_body(s, c, wp_r, bp_r, wo_r, bo_r, o):
    p = s[...] / jnp.maximum(c[...], 1.0)
    t = jnp.dot(p, wp_r[...], preferred_element_type=jnp.float32) + bp_r[...]
    h = jnp.maximum(t, 0.0) + jnp.log1p(jnp.exp(-jnp.abs(t)))
    o[...] = jnp.dot(h, wo_r[...],
                     preferred_element_type=jnp.float32) + bo_r[...]


def _pad_edges(ei):
    e = ei.shape[1]
    ep = _ceil_to(e, _CH * _NTS)
    src = ei[0]
    dst = ei[1]
    if ep != e:
        src = jnp.concatenate([src, jnp.zeros((ep - e,), jnp.int32)])
        dst = jnp.concatenate([dst, jnp.full((ep - e,), -1, jnp.int32)])
    return src, dst


def _pad_rows(x, n):
    return jnp.pad(x, ((0, n - x.shape[0]), (0, 0)))


def kernel(x_atom, x_bond, x_triplet, x_motif, edge_index_motif_motif,
           edge_index_motif_triplet, edge_index_triplet_triplet,
           edge_index_triplet_bond, edge_index_bond_bond,
           edge_index_bond_atom, edge_index_atom_atom, batch,
           Wl, Wr, b, Wp, bp, Wo, bo):
    na, nb, nt, nm = (x_atom.shape[0], x_bond.shape[0],
                      x_triplet.shape[0], x_motif.shape[0])
    ng = 1024
    np_a, np_b, np_t, np_m = (_np_rows(na), _np_rows(nb),
                              _np_rows(nt), _np_rows(nm))

    xa, xb, xt, xm = (_pad_rows(x_atom, np_a), _pad_rows(x_bond, np_b),
                      _pad_rows(x_triplet, np_t), _pad_rows(x_motif, np_m))
    e_mm = _pad_edges(edge_index_motif_motif)
    e_mt = _pad_edges(edge_index_motif_triplet)
    e_tt = _pad_edges(edge_index_triplet_triplet)
    e_tb = _pad_edges(edge_index_triplet_bond)
    e_bb = _pad_edges(edge_index_bond_bond)
    e_ba = _pad_edges(edge_index_bond_atom)
    e_aa = _pad_edges(edge_index_atom_atom)

    segs = {}

    def seg(type_list, args, with_cnt):
        key = (tuple(type_list), with_cnt)
        if key not in segs:
            segs[key] = _seg_builder(tuple(type_list), with_cnt)
        return segs[key](*args)

    upds = {}

    def upd(np_rows, n_agg, *args):
        key = (np_rows, n_agg)
        if key not in upds:
            upds[key] = _upd_builder(np_rows, n_agg)
        return upds[key](*args)

    layer_types = [
        (np_m, nm, e_mm[0].shape[0]), (np_m, nt, e_mt[0].shape[0]),
        (np_t, nt, e_tt[0].shape[0]), (np_t, nb, e_tb[0].shape[0]),
        (np_b, nb, e_bb[0].shape[0]), (np_b, na, e_ba[0].shape[0]),
        (np_a, na, e_aa[0].shape[0]),
    ]
    cnts = {}
    for l in range(3):
        first = l == 0
        args = (e_mm[0], e_mm[1], xm, e_mt[0], e_mt[1], xm,
                e_tt[0], e_tt[1], xt, e_tb[0], e_tb[1], xt,
                e_bb[0], e_bb[1], xb, e_ba[0], e_ba[1], xb,
                e_aa[0], e_aa[1], xa)
        r = seg(layer_types, args, first)
        if first:
            for i, name in enumerate(("mm", "mt", "tt", "tb", "bb",
                                      "ba", "aa")):
                cnts[name] = r[7 + i].reshape(-1, 1)
        s_mm, s_mt, s_tt, s_tb, s_bb, s_ba, s_aa = r[:7]

        xm_new = upd(np_m, 1, s_mm, cnts["mm"], xm, Wl[l, 0], Wr[l, 0],
                     b[l, 0].reshape(1, _H))
        xt_new = upd(np_t, 2, s_mt, cnts["mt"], s_tt, cnts["tt"], xt,
                     Wl[l, 1], Wl[l, 2], Wr[l, 1], Wr[l, 2],
                     b[l, 1].reshape(1, _H), b[l, 2].reshape(1, _H))
        xb_new = upd(np_b, 2, s_tb, cnts["tb"], s_bb, cnts["bb"], xb,
                     Wl[l, 3], Wl[l, 4], Wr[l, 3], Wr[l, 4],
                     b[l, 3].reshape(1, _H), b[l, 4].reshape(1, _H))
        xa_new = upd(np_a, 2, s_ba, cnts["ba"], s_aa, cnts["aa"], xa,
                     Wl[l, 5], Wl[l, 6], Wr[l, 5], Wr[l, 6],
                     b[l, 5].reshape(1, _H), b[l, 6].reshape(1, _H))
        xm, xt, xb, xa = xm_new, xt_new, xb_new, xa_new

    e_pool = _pad_edges(jnp.stack([jnp.arange(na, dtype=jnp.int32), batch]))
    ps, pc = seg([(np_a, ng, e_pool[0].shape[0])],
                 (e_pool[0], e_pool[1], xa), True)

    wo_pad = jnp.pad(Wo, ((0, 0), (0, 127)))
    bo_pad = jnp.pad(bo.reshape(1, 1), ((0, 0), (0, 127)))
    head = pl.pallas_call(
        _head_body,
        out_shape=jax.ShapeDtypeStruct((ng, 128), jnp.float32),
    )(ps, pc.reshape(-1, 1), Wp, bp.reshape(1, _H), wo_pad, bo_pad)
    return head[:, :1]


# speedup vs baseline: 1.7290x; 1.7290x over previous
import jax
import jax.numpy as jnp
from jax import lax
from jax.experimental import pallas as pl
from jax.experimental.pallas import tpu as pltpu
from jax.experimental.pallas import tpu_sc as plsc

_H = 64
_CH = 4096
_SUB = 256
_NTS = 16


def _ceil_to(x, m):
    return -(-x // m) * m


def _seg_cfg(n_dst):
    half = -(-n_dst // 2)
    n_pass = -(-half // 16384)
    nq = _ceil_to(-(-n_dst // (2 * n_pass)), 512)
    return n_pass, nq, 2 * n_pass * nq


def _np_rows(n_dst):
    return _seg_cfg(n_dst)[2]


def _seg_builder(n_src_pad, n_dst, n_ep, with_cnt):
    n_pass, nq, np_out = _seg_cfg(n_dst)
    accr = _ceil_to(nq + 16, _SUB)
    n_chunks_per_tile = n_ep // (_CH * _NTS)
    nzc = accr // _SUB
    noc = nq // _SUB

    mesh = plsc.VectorSubcoreMesh(core_axis_name="c", subcore_axis_name="s")

    out_type = [jax.ShapeDtypeStruct((np_out, _H), jnp.float32)]
    if with_cnt:
        out_type.append(jax.ShapeDtypeStruct((np_out,), jnp.float32))

    nsr = _CH // _SUB + 1
    scratch = [
        pltpu.VMEM((_CH,), jnp.int32),
        pltpu.VMEM((_CH,), jnp.int32),
        pltpu.VMEM((nsr, _SUB), jnp.int32),
        pltpu.VMEM((nsr, _SUB), jnp.int32),
        pltpu.VMEM((_SUB, _H), jnp.float32),
        pltpu.VMEM((_SUB, _H), jnp.float32),
        pltpu.VMEM((_SUB, _H), jnp.float32),
        pltpu.VMEM((_SUB,), jnp.float32),
        pltpu.VMEM((_SUB,), jnp.float32),
        pltpu.VMEM_SHARED((accr, _H), jnp.float32),
        pltpu.VMEM_SHARED((accr,), jnp.float32),
        pltpu.SemaphoreType.DMA,
        pltpu.SemaphoreType.DMA,
        pltpu.SemaphoreType.DMA,
        pltpu.SemaphoreType.DMA,
    ]

    def body(src_ref, dst_ref, x_ref, *rest):
        if with_cnt:
            out_ref, cnt_ref = rest[0], rest[1]
            scr = rest[2:]
        else:
            out_ref, cnt_ref = rest[0], None
            scr = rest[1:]
        (s_src, s_dst, k_src, k_dst, rows0, rows1, zrows, zvec, ones,
         acc, cacc, sg0, sg1, ss0, ss1) = scr
        rows_b = (rows0, rows1)
        sg_b = (sg0, sg1)
        ss_b = (ss0, ss1)
        c = lax.axis_index("c")
        s = lax.axis_index("s")

        zero16 = jnp.zeros((16,), jnp.float32)
        one16 = jnp.ones((16,), jnp.float32)

        def zrow_body(i, _):
            for jj in range(_H // 16):
                zrows[i, pl.ds(jj * 16, 16)] = zero16
            return 0
        lax.fori_loop(0, _SUB, zrow_body, 0)

        def zvec_body(i, _):
            zvec[pl.ds(i * 16, 16)] = zero16
            ones[pl.ds(i * 16, 16)] = one16
            return 0
        lax.fori_loop(0, _SUB // 16, zvec_body, 0)

        di = lax.iota(jnp.int32, 16)
        dump_d = nq + di
        dump_s = s * 16 + di

        def one_pass(p):
            q = 2 * p + c
            lo = q * nq
            hi = jnp.minimum(lo + nq, n_dst)

            for k in range(-(-nzc // _NTS)):
                ci = k * _NTS + s

                @pl.when(ci < nzc)
                def _():
                    pltpu.sync_copy(zrows, acc.at[pl.ds(ci * _SUB, _SUB)])
                    if with_cnt:
                        pltpu.sync_copy(zvec, cacc.at[pl.ds(ci * _SUB, _SUB)])
            plsc.subcore_barrier()

            def chunk_body(ci, _):
                base = (ci * _NTS + s) * _CH
                pltpu.sync_copy(src_ref.at[pl.ds(base, _CH)], s_src)
                pltpu.sync_copy(dst_ref.at[pl.ds(base, _CH)], s_dst)

                def pf(i, _):
                    r = i // (_SUB // 16)
                    g = i % (_SUB // 16)
                    k_src[r, pl.ds(g * 16, 16)] = dump_s
                    k_dst[r, pl.ds(g * 16, 16)] = dump_d
                    return 0
                lax.fori_loop(0, nsr * (_SUB // 16), pf, 0)

                def fl(i, kc):
                    d = s_dst[pl.ds(i * 16, 16)]
                    sr = s_src[pl.ds(i * 16, 16)]
                    m = (d >= lo) & (d < hi)
                    mi = m.astype(jnp.int32)
                    pos = kc + plsc.cumsum(mi) - mi
                    pr = pos // _SUB
                    pc = pos % _SUB
                    plsc.store_scatter(k_src, [pr, pc], sr, mask=m)
                    plsc.store_scatter(k_dst, [pr, pc], d - lo, mask=m)
                    return kc + jnp.sum(mi)
                kc = lax.fori_loop(0, _CH // 16, fl, jnp.int32(0))
                nsub = (kc + _SUB - 1) // _SUB

                @pl.when(nsub > 0)
                def _():
                    pltpu.async_copy(x_ref.at[k_src.at[0]], rows0, sg0)

                def sub2(t, _):
                    for kb in range(2):
                        j = 2 * t + kb
                        rw, sg = rows_b[kb], sg_b[kb]
                        rw_o, sg_o = rows_b[1 - kb], sg_b[1 - kb]

                        @pl.when(j < nsub)
                        def _():
                            @pl.when(j + 1 < nsub)
                            def _():
                                pltpu.async_copy(
                                    x_ref.at[k_src.at[j + 1]], rw_o, sg_o)
                            pltpu.make_async_copy(
                                x_ref.at[k_src.at[j]], rw, sg).wait()
                            pltpu.sync_copy(rw, acc.at[k_dst.at[j]],
                                            add=True)
                            if with_cnt:
                                pltpu.sync_copy(ones, cacc.at[k_dst.at[j]],
                                                add=True)
                    return 0
                lax.fori_loop(0, (nsub + 1) // 2, sub2, 0)
                return 0
            lax.fori_loop(0, n_chunks_per_tile, chunk_body, 0)

            plsc.subcore_barrier()

            for k in range(-(-noc // _NTS)):
                ci = k * _NTS + s

                @pl.when(ci < noc)
                def _():
                    off = ci * _SUB
                    pltpu.sync_copy(acc.at[pl.ds(off, _SUB)],
                                    out_ref.at[pl.ds(lo + off, _SUB)])
                    if with_cnt:
                        pltpu.sync_copy(cacc.at[pl.ds(off, _SUB)],
                                        cnt_ref.at[pl.ds(lo + off, _SUB)])
            plsc.subcore_barrier()

        for p in range(n_pass):
            one_pass(p)

    return pl.kernel(body, out_type=out_type, mesh=mesh,
                     scratch_types=scratch,
                     compiler_params=pltpu.CompilerParams(
                         needs_layout_passes=False,
                         use_tc_tiling_on_sc=False))


def _upd_builder(np_rows, n_agg):
    blk = 1024
    grid = (np_rows // blk,)

    def body(*refs):
        if n_agg == 2:
            s1, c1, s2, c2, x, w1, w2, wr1, wr2, b1, b2, o = refs
        else:
            s1, c1, x, w1, wr1, b1, o = refs
        xv = x[...]
        t = jnp.dot(s1[...] / jnp.maximum(c1[...], 1.0), w1[...],
                    preferred_element_type=jnp.float32)
        t = t + jnp.dot(xv, wr1[...], preferred_element_type=jnp.float32)
        t = t + b1[...]
        if n_agg == 2:
            t2 = jnp.dot(s2[...] / jnp.maximum(c2[...], 1.0), w2[...],
                         preferred_element_type=jnp.float32)
            t2 = t2 + jnp.dot(xv, wr2[...],
                              preferred_element_type=jnp.float32)
            t = t + (t2 + b2[...])
        o[...] = jnp.maximum(t, 0.0)

    bs_feat = pl.BlockSpec((blk, _H), lambda i: (i, 0))
    bs_cnt = pl.BlockSpec((blk, 1), lambda i: (i, 0))
    bs_w = pl.BlockSpec((_H, _H), lambda i: (0, 0))
    bs_b = pl.BlockSpec((1, _H), lambda i: (0, 0))
    if n_agg == 2:
        in_specs = [bs_feat, bs_cnt, bs_feat, bs_cnt, bs_feat,
                    bs_w, bs_w, bs_w, bs_w, bs_b, bs_b]
    else:
        in_specs = [bs_feat, bs_cnt, bs_feat, bs_w, bs_w, bs_b]
    return pl.pallas_call(
        body, grid=grid, in_specs=in_specs,
        out_specs=pl.BlockSpec((blk, _H), lambda i: (i, 0)),
        out_shape=jax.ShapeDtypeStruct((np_rows, _H), jnp.float32))


def _head_body(s, c, wp_r, bp_r, wo_r, bo_r, o):
    p = s[...] / jnp.maximum(c[...], 1.0)
    t = jnp.dot(p, wp_r[...], preferred_element_type=jnp.float32) + bp_r[...]
    h = jnp.maximum(t, 0.0) + jnp.log1p(jnp.exp(-jnp.abs(t)))
    o[...] = jnp.dot(h, wo_r[...],
                     preferred_element_type=jnp.float32) + bo_r[...]


def _pad_edges(ei):
    e = ei.shape[1]
    ep = _ceil_to(e, _CH * _NTS)
    src = ei[0]
    dst = ei[1]
    if ep != e:
        src = jnp.concatenate([src, jnp.zeros((ep - e,), jnp.int32)])
        dst = jnp.concatenate([dst, jnp.full((ep - e,), -1, jnp.int32)])
    return src, dst


def _pad_rows(x, n):
    return jnp.pad(x, ((0, n - x.shape[0]), (0, 0)))


def kernel(x_atom, x_bond, x_triplet, x_motif, edge_index_motif_motif,
           edge_index_motif_triplet, edge_index_triplet_triplet,
           edge_index_triplet_bond, edge_index_bond_bond,
           edge_index_bond_atom, edge_index_atom_atom, batch,
           Wl, Wr, b, Wp, bp, Wo, bo):
    na, nb, nt, nm = (x_atom.shape[0], x_bond.shape[0],
                      x_triplet.shape[0], x_motif.shape[0])
    ng = 1024
    np_a, np_b, np_t, np_m = (_np_rows(na), _np_rows(nb),
                              _np_rows(nt), _np_rows(nm))

    xa, xb, xt, xm = (_pad_rows(x_atom, np_a), _pad_rows(x_bond, np_b),
                      _pad_rows(x_triplet, np_t), _pad_rows(x_motif, np_m))
    e_mm = _pad_edges(edge_index_motif_motif)
    e_mt = _pad_edges(edge_index_motif_triplet)
    e_tt = _pad_edges(edge_index_triplet_triplet)
    e_tb = _pad_edges(edge_index_triplet_bond)
    e_bb = _pad_edges(edge_index_bond_bond)
    e_ba = _pad_edges(edge_index_bond_atom)
    e_aa = _pad_edges(edge_index_atom_atom)

    segs = {}

    def seg(e_pair, x_p, n_dst, with_cnt):
        key = (x_p.shape[0], n_dst, e_pair[0].shape[0], with_cnt)
        if key not in segs:
            segs[key] = _seg_builder(*key)
        return segs[key](e_pair[0], e_pair[1], x_p)

    upds = {}

    def upd(np_rows, n_agg, *args):
        key = (np_rows, n_agg)
        if key not in upds:
            upds[key] = _upd_builder(np_rows, n_agg)
        return upds[key](*args)

    cnts = {}
    for l in range(3):
        first = l == 0
        r_mm = seg(e_mm, xm, nm, first)
        r_mt = seg(e_mt, xm, nt, first)
        r_tt = seg(e_tt, xt, nt, first)
        r_tb = seg(e_tb, xt, nb, first)
        r_bb = seg(e_bb, xb, nb, first)
        r_ba = seg(e_ba, xb, na, first)
        r_aa = seg(e_aa, xa, na, first)
        if first:
            for name, r in (("mm", r_mm), ("mt", r_mt), ("tt", r_tt),
                            ("tb", r_tb), ("bb", r_bb), ("ba", r_ba),
                            ("aa", r_aa)):
                cnts[name] = r[1].reshape(-1, 1)
        s_mm, s_mt, s_tt = r_mm[0], r_mt[0], r_tt[0]
        s_tb, s_bb, s_ba, s_aa = r_tb[0], r_bb[0], r_ba[0], r_aa[0]

        xm_new = upd(np_m, 1, s_mm, cnts["mm"], xm, Wl[l, 0], Wr[l, 0],
                     b[l, 0].reshape(1, _H))
        xt_new = upd(np_t, 2, s_mt, cnts["mt"], s_tt, cnts["tt"], xt,
                     Wl[l, 1], Wl[l, 2], Wr[l, 1], Wr[l, 2],
                     b[l, 1].reshape(1, _H), b[l, 2].reshape(1, _H))
        xb_new = upd(np_b, 2, s_tb, cnts["tb"], s_bb, cnts["bb"], xb,
                     Wl[l, 3], Wl[l, 4], Wr[l, 3], Wr[l, 4],
                     b[l, 3].reshape(1, _H), b[l, 4].reshape(1, _H))
        xa_new = upd(np_a, 2, s_ba, cnts["ba"], s_aa, cnts["aa"], xa,
                     Wl[l, 5], Wl[l, 6], Wr[l, 5], Wr[l, 6],
                     b[l, 5].reshape(1, _H), b[l, 6].reshape(1, _H))
        xm, xt, xb, xa = xm_new, xt_new, xb_new, xa_new

    e_pool = _pad_edges(jnp.stack([jnp.arange(na, dtype=jnp.int32), batch]))
    ps, pc = seg(e_pool, xa, ng, True)

    wo_pad = jnp.pad(Wo, ((0, 0), (0, 127)))
    bo_pad = jnp.pad(bo.reshape(1, 1), ((0, 0), (0, 127)))
    head = pl.pallas_call(
        _head_body,
        out_shape=jax.ShapeDtypeStruct((ng, 128), jnp.float32),
    )(ps, pc.reshape(-1, 1), Wp, bp.reshape(1, _H), wo_pad, bo_pad)
    return head[:, :1]
